# baseline (speedup 1.0000x reference)
import functools

import jax
import jax.numpy as jnp
from jax import lax
from jax.experimental import pallas as pl
from jax.experimental.pallas import tpu as pltpu
from jax.experimental.pallas import tpu_sc as plsc

_L = 16


def _shuffle(v, perm):
  dn = lax.GatherDimensionNumbers(
      offset_dims=(), collapsed_slice_dims=(0,), start_index_map=(0,))
  return lax.gather(v, perm[:, None], dn, slice_sizes=(1,),
                    mode=lax.GatherScatterMode.PROMISE_IN_BOUNDS)


def _emb_ln_kernel(n_rows, d_model, seq_len, rows_per_w, chunk):
  nsl = d_model // _L
  n_chunks = rows_per_w // chunk
  mesh = plsc.VectorSubcoreMesh(core_axis_name="c", subcore_axis_name="s")

  @functools.partial(
      pl.kernel,
      out_type=jax.ShapeDtypeStruct((n_rows, d_model), jnp.float32),
      mesh=mesh,
      scratch_types=[
          pltpu.VMEM((rows_per_w,), jnp.int32),
          pltpu.VMEM((chunk, d_model), jnp.float32),
          pltpu.VMEM((chunk, d_model), jnp.float32),
          pltpu.VMEM((d_model,), jnp.float32),
          pltpu.VMEM((d_model,), jnp.float32),
          pltpu.SemaphoreType.DMA,
      ],
  )
  def body(x_hbm, w_hbm, p_hbm, g_hbm, b_hbm, out_hbm,
           idx_v, hbuf, pbuf, g_v, b_v, sem):
    wid = lax.axis_index("s") * 2 + lax.axis_index("c")
    base = wid * rows_per_w
    s_start = lax.rem(base, seq_len)
    pltpu.sync_copy(x_hbm.at[pl.ds(base, rows_per_w)], idx_v)
    pltpu.sync_copy(g_hbm, g_v)
    pltpu.sync_copy(b_hbm, b_v)

    def do_chunk(g):
      pltpu.async_copy(w_hbm.at[idx_v.at[pl.ds(g * chunk, chunk)]], hbuf,
                       sem).wait()
      pltpu.sync_copy(p_hbm.at[pl.ds(s_start + g * chunk, chunk)], pbuf)

      def row_body(r, carry):
        vsum = jnp.zeros((_L,), jnp.float32)
        vsq = jnp.zeros((_L,), jnp.float32)
        for j in range(nsl):
          sl = pl.ds(j * _L, _L)
          h = hbuf[r, sl] + pbuf[r, sl]
          hbuf[r, sl] = h
          vsum = vsum + h
          vsq = vsq + h * h
        for sh in (1, 2, 4, 8):
          perm = jnp.arange(_L, dtype=jnp.int32) ^ sh
          vsum = vsum + _shuffle(vsum, perm)
          vsq = vsq + _shuffle(vsq, perm)
        mean_v = vsum * (1.0 / d_model)
        var_v = vsq * (1.0 / d_model) - mean_v * mean_v
        xv = jnp.maximum(var_v, 0.0) + 1e-5
        m = xv * 262144.0
        r = jnp.full((_L,), 512.0, jnp.float32)
        for p in (16, 8, 4, 2, 1):
          c = m >= jnp.float32(4.0 ** p)
          m = jnp.where(c, m * jnp.float32(4.0 ** -p), m)
          scl = jnp.where(c, scl * jnp.float32(2.0 ** -p), scl)
        y = 0.5 + 0.5 / m
        half = 0.5 * m
        for _ in range(4):
          y = y * (1.5 - half * y * y)
        rs_v = y * scl
        for j in range(nsl):
          sl = pl.ds(j * _L, _L)
          h = hbuf[r, sl]
          hbuf[r, sl] = (h - mean_v) * (g_v[sl] * rs_v) + b_v[sl]
        return carry

      lax.fori_loop(0, chunk, row_body, 0)
      pltpu.sync_copy(hbuf, out_hbm.at[pl.ds(base + g * chunk, chunk)])

    for g in range(n_chunks):
      do_chunk(g)

  return body


def kernel(x, W, P, gamma, beta):
  b, s = x.shape
  vocab, d = W.shape
  n = b * s
  xf = x.reshape(n).astype(jnp.int32)
  n_workers = 32
  rows_per_w = n // n_workers
  chunk = min(64, rows_per_w)
  fn = _emb_ln_kernel(n, d, s, rows_per_w, chunk)
  out = fn(xf, W, P, gamma, beta)
  return out.reshape(b, s, d)


# speedup vs baseline: 1.0748x; 1.0748x over previous
import functools

import jax
import jax.numpy as jnp
from jax import lax
from jax.experimental import pallas as pl
from jax.experimental.pallas import tpu as pltpu
from jax.experimental.pallas import tpu_sc as plsc

_L = 16


def _shuffle(v, perm):
  dn = lax.GatherDimensionNumbers(
      offset_dims=(), collapsed_slice_dims=(0,), start_index_map=(0,))
  return lax.gather(v, perm[:, None], dn, slice_sizes=(1,),
                    mode=lax.GatherScatterMode.PROMISE_IN_BOUNDS)


def _emb_ln_kernel(n_rows, d_model, seq_len, rows_per_w, chunk):
  nsl = d_model // _L
  n_chunks = rows_per_w // chunk
  mesh = plsc.VectorSubcoreMesh(core_axis_name="c", subcore_axis_name="s")

  @functools.partial(
      pl.kernel,
      out_type=jax.ShapeDtypeStruct((n_rows, d_model), jnp.float32),
      mesh=mesh,
      scratch_types=[
          pltpu.VMEM((rows_per_w,), jnp.int32),
          pltpu.VMEM((2, chunk, d_model), jnp.float32),
          pltpu.VMEM((2, chunk, d_model), jnp.float32),
          pltpu.VMEM((d_model,), jnp.float32),
          pltpu.VMEM((d_model,), jnp.float32),
          pltpu.SemaphoreType.DMA,
          pltpu.SemaphoreType.DMA,
          pltpu.SemaphoreType.DMA,
          pltpu.SemaphoreType.DMA,
      ],
  )
  def body(x_hbm, w_hbm, p_hbm, g_hbm, b_hbm, out_hbm,
           idx_v, hbuf2, pbuf2, g_v, b_v, sem_in0, sem_in1, sem_out0,
           sem_out1):
    wid = lax.axis_index("s") * 2 + lax.axis_index("c")
    base = wid * rows_per_w
    s_start = lax.rem(base, seq_len)
    pltpu.sync_copy(x_hbm.at[pl.ds(base, rows_per_w)], idx_v)
    pltpu.sync_copy(g_hbm, g_v)
    pltpu.sync_copy(b_hbm, b_v)
    sem_in = (sem_in0, sem_in1)
    sem_out = (sem_out0, sem_out1)

    def start_in(g):
      b = g & 1
      return (
          pltpu.async_copy(w_hbm.at[idx_v.at[pl.ds(g * chunk, chunk)]],
                           hbuf2.at[b], sem_in[b]),
          pltpu.async_copy(p_hbm.at[pl.ds(s_start + g * chunk, chunk)],
                           pbuf2.at[b], sem_in[b]),
      )

    def compute_chunk(g):
      b = g & 1
      hbuf = hbuf2.at[b]
      pbuf = pbuf2.at[b]

      def row_body(r, carry):
        vsum = jnp.zeros((_L,), jnp.float32)
        vsq = jnp.zeros((_L,), jnp.float32)
        for j in range(nsl):
          sl = pl.ds(j * _L, _L)
          h = hbuf[r, sl] + pbuf[r, sl]
          hbuf[r, sl] = h
          vsum = vsum + h
          vsq = vsq + h * h
        for sh in (1, 2, 4, 8):
          perm = jnp.arange(_L, dtype=jnp.int32) ^ sh
          vsum = vsum + _shuffle(vsum, perm)
          vsq = vsq + _shuffle(vsq, perm)
        mean_v = vsum * (1.0 / d_model)
        var_v = vsq * (1.0 / d_model) - mean_v * mean_v
        xv = jnp.maximum(var_v, 0.0) + 1e-5
        m = xv * 262144.0
        r = jnp.full((_L,), 512.0, jnp.float32)
        for p in (16, 8, 4, 2, 1):
          c = m >= jnp.float32(4.0 ** p)
          m = jnp.where(c, m * jnp.float32(4.0 ** -p), m)
          scl = jnp.where(c, scl * jnp.float32(2.0 ** -p), scl)
        y = 0.5 + 0.5 / m
        half = 0.5 * m
        for _ in range(4):
          y = y * (1.5 - half * y * y)
        rs_v = y * scl
        for j in range(nsl):
          sl = pl.ds(j * _L, _L)
          h = hbuf[r, sl]
          hbuf[r, sl] = (h - mean_v) * (g_v[sl] * rs_v) + b_v[sl]
        return carry

      lax.fori_loop(0, chunk, row_body, 0)
      return pltpu.async_copy(hbuf, out_hbm.at[pl.ds(base + g * chunk, chunk)],
                              sem_out[b])

    in_h = [None, None]
    out_h = [None, None]
    in_h[0] = start_in(0)
    for g in range(n_chunks):
      b = g & 1
      if g + 1 < n_chunks:
        if out_h[1 - b] is not None:
          out_h[1 - b].wait()
        in_h[1 - b] = start_in(g + 1)
      for h in in_h[b]:
        h.wait()
      out_h[b] = compute_chunk(g)
    if n_chunks > 1:
      out_h[(n_chunks - 2) & 1].wait()
    out_h[(n_chunks - 1) & 1].wait()

  return body


def kernel(x, W, P, gamma, beta):
  b, s = x.shape
  vocab, d = W.shape
  n = b * s
  xf = x.reshape(n).astype(jnp.int32)
  n_workers = 32
  rows_per_w = n // n_workers
  chunk = min(32, rows_per_w)
  fn = _emb_ln_kernel(n, d, s, rows_per_w, chunk)
  out = fn(xf, W, P, gamma, beta)
  return out.reshape(b, s, d)
